# baseline (speedup 1.0000x reference)
import functools

import jax
import jax.numpy as jnp
from jax import lax
from jax.experimental import pallas as pl
from jax.experimental.pallas import tpu as pltpu
from jax.experimental.pallas import tpu_sc as plsc

N = 10000
E = 160000
D = 256
C = 64
EB = 128
EPAD = 163840
NPAD = 10240
HALF = 5000
APAD = 5248
DUMMY = 5100
RPT_DEG = NPAD // 16
RPT = APAD // 16


@functools.cache
def _make_deg():
    mesh = plsc.VectorSubcoreMesh(core_axis_name="c", subcore_axis_name="s")
    nb = EPAD // 32 // EB

    @functools.partial(
        pl.kernel,
        out_type=jax.ShapeDtypeStruct((2, NPAD, 128), jnp.float32),
        mesh=mesh,
        scratch_types=[
            pltpu.VMEM((nb, EB), jnp.int32),
            pltpu.VMEM((EB, 128), jnp.float32),
            pltpu.VMEM_SHARED((NPAD, 128), jnp.float32),
        ],
    )
    def deg_kernel(dst_hbm, ones_hbm, zeros_hbm, out_hbm, dst_v, ones_v, acc):
        c = lax.axis_index("c")
        s = lax.axis_index("s")
        wid = s * 2 + c
        pltpu.sync_copy(dst_hbm.at[wid], dst_v)
        pltpu.sync_copy(ones_hbm, ones_v)
        pltpu.sync_copy(zeros_hbm, acc.at[pl.ds(s * RPT_DEG, RPT_DEG)])
        plsc.subcore_barrier()

        def body(b, carry):
            pltpu.sync_copy(ones_v, acc.at[dst_v.at[b]], add=True)
            return carry

        lax.fori_loop(0, nb, body, 0)
        plsc.subcore_barrier()
        sl = pl.ds(s * RPT_DEG, RPT_DEG)
        pltpu.sync_copy(acc.at[sl], out_hbm.at[c, sl])

    return deg_kernel


@functools.cache
def _make_spmm(W, edge_split):
    nb = EPAD // (32 if edge_split else 16) // EB
    nb_hi, nb_lo = 52, 28
    mesh = plsc.VectorSubcoreMesh(core_axis_name="c", subcore_axis_name="s")

    @functools.partial(
        pl.kernel,
        out_type=jax.ShapeDtypeStruct((2, NPAD, W), jnp.float32),
        mesh=mesh,
        scratch_types=[
            pltpu.VMEM((nb_hi if edge_split else nb, EB), jnp.int32),
            pltpu.VMEM((nb_hi if edge_split else nb, EB), jnp.int32),
            pltpu.VMEM((EB, W), jnp.float32),
            pltpu.VMEM_SHARED((NPAD, W), jnp.float32),
        ],
        compiler_params=pltpu.CompilerParams(use_tc_tiling_on_sc=(W == 128)),
    )
    def spmm(hs_hbm, src_hbm, dst_hbm, zeros_hbm, out_hbm, src_v, dst_v, buf,
             acc):
        c = lax.axis_index("c")
        s = lax.axis_index("s")
        if edge_split:
            wid = s * 2 + c
            pltpu.sync_copy(src_hbm.at[wid], src_v)
            pltpu.sync_copy(dst_hbm.at[wid], dst_v)
            nloop = lax.select(c == 1, nb_hi, nb_lo)
        else:
            pltpu.sync_copy(src_hbm.at[c, s], src_v)
            pltpu.sync_copy(dst_hbm.at[s], dst_v)
            nloop = nb
        sl = pl.ds(s * RPT_DEG, RPT_DEG)
        pltpu.sync_copy(zeros_hbm, acc.at[sl])
        plsc.subcore_barrier()

        def body(b, carry):
            pltpu.sync_copy(hs_hbm.at[src_v.at[b]], buf)
            pltpu.sync_copy(buf, acc.at[dst_v.at[b]], add=True)
            return carry

        lax.fori_loop(0, nloop, body, 0)
        plsc.subcore_barrier()
        pltpu.sync_copy(acc.at[sl], out_hbm.at[c, sl])

    return spmm


RB = 1000
GRID = N // RB


def _tc1_body(x_ref, w_ref, degp_ref, o_ref, dinv_ref):
    dinv = lax.rsqrt(degp_ref[0] + degp_ref[1] + 1.0)
    dinv_ref[...] = dinv
    h = jnp.dot(x_ref[...], w_ref[...], preferred_element_type=jnp.float32)
    hs = h * dinv[:, 0:1]
    o_ref[0] = hs[:, :128]
    o_ref[1] = hs[:, 128:]


def _tc1(x, w1, degp):
    return pl.pallas_call(
        _tc1_body,
        grid=(GRID,),
        in_specs=[
            pl.BlockSpec((RB, D), lambda i: (i, 0)),
            pl.BlockSpec((D, D), lambda i: (0, 0)),
            pl.BlockSpec((2, RB, 128), lambda i: (0, i, 0)),
        ],
        out_specs=[
            pl.BlockSpec((2, RB, 128), lambda i: (0, i, 0)),
            pl.BlockSpec((RB, 128), lambda i: (i, 0)),
        ],
        out_shape=[
            jax.ShapeDtypeStruct((2, N, 128), jnp.float32),
            jax.ShapeDtypeStruct((N, 128), jnp.float32),
        ],
    )(x, w1, degp)


def _tc_mid_body(dn, s_ref, hs_ref, dinv_ref, b_ref, w_ref, o_ref):
    d = dinv_ref[...]
    z0 = jax.nn.relu((s_ref[0] + hs_ref[0]) * d + b_ref[0, :128])
    z1 = jax.nn.relu((s_ref[1] + hs_ref[1]) * d + b_ref[0, 128:])
    z = jnp.concatenate([z0, z1], axis=1)
    h = jnp.dot(z, w_ref[...], preferred_element_type=jnp.float32)
    hs = h * d[:, 0:1]
    if dn == D:
        o_ref[0] = hs[:, :128]
        o_ref[1] = hs[:, 128:]
    else:
        o_ref[...] = hs


def _tc_mid(s_part, hs_prev, dinv_b, b_vec, w_next):
    dn = w_next.shape[1]
    if dn == D:
        out_spec = pl.BlockSpec((2, RB, 128), lambda i: (0, i, 0))
        out_shape = jax.ShapeDtypeStruct((2, N, 128), jnp.float32)
    else:
        out_spec = pl.BlockSpec((RB, dn), lambda i: (i, 0))
        out_shape = jax.ShapeDtypeStruct((N, dn), jnp.float32)
    return pl.pallas_call(
        functools.partial(_tc_mid_body, dn),
        grid=(GRID,),
        in_specs=[
            pl.BlockSpec((2, RB, 128), lambda i: (0, i, 0)),
            pl.BlockSpec((2, RB, 128), lambda i: (0, i, 0)),
            pl.BlockSpec((RB, 128), lambda i: (i, 0)),
            pl.BlockSpec((1, D), lambda i: (0, 0)),
            pl.BlockSpec((D, dn), lambda i: (0, 0)),
        ],
        out_specs=out_spec,
        out_shape=out_shape,
    )(s_part, hs_prev, dinv_b, b_vec, w_next)


def _tc_out_body(s_ref, hs_ref, dinv_ref, b_ref, o_ref):
    val = (s_ref[0] + s_ref[1] + hs_ref[...]) * dinv_ref[:, 0:1]
    o_ref[...] = val + b_ref[0, :]


def _tc_out(s3, hs3, dinv_b, b3):
    return pl.pallas_call(
        _tc_out_body,
        grid=(GRID,),
        in_specs=[
            pl.BlockSpec((2, RB, C), lambda i: (0, i, 0)),
            pl.BlockSpec((RB, C), lambda i: (i, 0)),
            pl.BlockSpec((RB, 128), lambda i: (i, 0)),
            pl.BlockSpec((1, C), lambda i: (0, 0)),
        ],
        out_specs=pl.BlockSpec((RB, C), lambda i: (i, 0)),
        out_shape=jax.ShapeDtypeStruct((N, C), jnp.float32),
    )(s3, hs3, dinv_b, b3)


def kernel(x, edge_index, W1, b1, W2, b2, W3, b3):
    src = edge_index[0].astype(jnp.int32)
    dst = edge_index[1].astype(jnp.int32)
    pad = EPAD - E
    src_p = jnp.concatenate([src, jnp.zeros((pad,), jnp.int32)])
    dst_p = jnp.concatenate([dst, jnp.full((pad,), N, jnp.int32)])
    src_fs = jnp.stack([src_p, src_p + N]).reshape(2, 16, EPAD // 16 // EB, EB)
    dst_fs = dst_p.reshape(16, EPAD // 16 // EB, EB)
    n0 = 16 * 52 * EB
    s0 = src_p[:n0].reshape(16, 52, EB)
    s1 = jnp.pad(src_p[n0:].reshape(16, 28, EB), ((0, 0), (0, 24), (0, 0)))
    d0 = dst_p[:n0].reshape(16, 52, EB)
    d1 = jnp.pad(dst_p[n0:].reshape(16, 28, EB), ((0, 0), (0, 24), (0, 0)),
                 constant_values=N)
    src_es = jnp.stack([s1, s0], axis=1).reshape(32, 52, EB)
    dst_es = jnp.stack([d1, d0], axis=1).reshape(32, 52, EB)
    dst_deg = dst_p.reshape(32, EPAD // 32 // EB, EB)

    ones_hbm = jnp.ones((EB, 128), jnp.float32)
    zeros_deg = jnp.zeros((RPT_DEG, 128), jnp.float32)
    zeros64 = jnp.zeros((RPT_DEG, C), jnp.float32)

    degp = _make_deg()(dst_deg, ones_hbm, zeros_deg)

    b1r = b1.reshape(1, D)
    b2r = b2.reshape(1, D)
    b3r = b3.reshape(1, C)


    spmm_fs = _make_spmm(128, False)
    spmm_es = _make_spmm(C, True)

    hs1, dinv_b = _tc1(x, W1, degp)
    s1 = spmm_fs(hs1.reshape(2 * N, 128), src_fs, dst_fs, zeros_deg)
    hs2 = _tc_mid(s1, hs1, dinv_b, b1r, W2)
    s2 = spmm_fs(hs2.reshape(2 * N, 128), src_fs, dst_fs, zeros_deg)
    hs3 = _tc_mid(s2, hs2, dinv_b, b2r, W3)
    s3 = spmm_es(hs3, src_es, dst_es, zeros64)
    return _tc_out(s3, hs3, dinv_b, b3r)


# speedup vs baseline: 1.0286x; 1.0286x over previous
import functools

import jax
import jax.numpy as jnp
from jax import lax
from jax.experimental import pallas as pl
from jax.experimental.pallas import tpu as pltpu
from jax.experimental.pallas import tpu_sc as plsc

N = 10000
E = 160000
D = 256
C = 64
EB = 128
EPAD = 163840
NPAD = 10240
HALF = 5000
APAD = 5248
DUMMY = 5100
RPT_DEG = NPAD // 16
RPT = APAD // 16


@functools.cache
def _make_deg():
    mesh = plsc.VectorSubcoreMesh(core_axis_name="c", subcore_axis_name="s")
    nb = EPAD // 32 // EB

    @functools.partial(
        pl.kernel,
        out_type=jax.ShapeDtypeStruct((2, NPAD, 128), jnp.float32),
        mesh=mesh,
        scratch_types=[
            pltpu.VMEM((nb, EB), jnp.int32),
            pltpu.VMEM((EB, 128), jnp.float32),
            pltpu.VMEM_SHARED((NPAD, 128), jnp.float32),
        ],
    )
    def deg_kernel(dst_hbm, ones_hbm, zeros_hbm, out_hbm, dst_v, ones_v, acc):
        c = lax.axis_index("c")
        s = lax.axis_index("s")
        wid = s * 2 + c
        pltpu.sync_copy(dst_hbm.at[wid], dst_v)
        pltpu.sync_copy(ones_hbm, ones_v)
        pltpu.sync_copy(zeros_hbm, acc.at[pl.ds(s * RPT_DEG, RPT_DEG)])
        plsc.subcore_barrier()

        def body(b, carry):
            pltpu.sync_copy(ones_v, acc.at[dst_v.at[b]], add=True)
            return carry

        lax.fori_loop(0, nb, body, 0)
        plsc.subcore_barrier()
        sl = pl.ds(s * RPT_DEG, RPT_DEG)
        pltpu.sync_copy(acc.at[sl], out_hbm.at[c, sl])

    return deg_kernel


@functools.cache
def _make_spmm(W, edge_split):
    nb = EPAD // (32 if edge_split else 16) // EB
    mesh = plsc.VectorSubcoreMesh(core_axis_name="c", subcore_axis_name="s")

    @functools.partial(
        pl.kernel,
        out_type=jax.ShapeDtypeStruct((2, NPAD, W), jnp.float32),
        mesh=mesh,
        scratch_types=[
            pltpu.VMEM((nb, EB), jnp.int32),
            pltpu.VMEM((nb, EB), jnp.int32),
            pltpu.VMEM((EB, W), jnp.float32),
            pltpu.VMEM_SHARED((NPAD, W), jnp.float32),
        ],
        compiler_params=pltpu.CompilerParams(use_tc_tiling_on_sc=False),
    )
    def spmm(hs_hbm, src_hbm, dst_hbm, zeros_hbm, out_hbm, src_v, dst_v, buf,
             acc):
        c = lax.axis_index("c")
        s = lax.axis_index("s")
        if edge_split:
            wid = s * 2 + c
            pltpu.sync_copy(src_hbm.at[wid], src_v)
            pltpu.sync_copy(dst_hbm.at[wid], dst_v)
        else:
            pltpu.sync_copy(src_hbm.at[c, s], src_v)
            pltpu.sync_copy(dst_hbm.at[s], dst_v)
        sl = pl.ds(s * RPT_DEG, RPT_DEG)
        pltpu.sync_copy(zeros_hbm, acc.at[sl])
        plsc.subcore_barrier()

        def body(b, carry):
            pltpu.sync_copy(hs_hbm.at[src_v.at[b]], buf)
            pltpu.sync_copy(buf, acc.at[dst_v.at[b]], add=True)
            return carry

        lax.fori_loop(0, nb, body, 0)
        plsc.subcore_barrier()
        pltpu.sync_copy(acc.at[sl], out_hbm.at[c, sl])

    return spmm


RB = 1000
GRID = N // RB


def _tc1_body(x_ref, w_ref, degp_ref, o_ref, dinv_ref):
    dinv = lax.rsqrt(degp_ref[0] + degp_ref[1] + 1.0)
    dinv_ref[...] = dinv
    h = jnp.dot(x_ref[...], w_ref[...], preferred_element_type=jnp.float32)
    hs = h * dinv[:, 0:1]
    o_ref[0] = hs[:, :128]
    o_ref[1] = hs[:, 128:]


def _tc1(x, w1, degp):
    return pl.pallas_call(
        _tc1_body,
        grid=(GRID,),
        in_specs=[
            pl.BlockSpec((RB, D), lambda i: (i, 0)),
            pl.BlockSpec((D, D), lambda i: (0, 0)),
            pl.BlockSpec((2, RB, 128), lambda i: (0, i, 0)),
        ],
        out_specs=[
            pl.BlockSpec((2, RB, 128), lambda i: (0, i, 0)),
            pl.BlockSpec((RB, 128), lambda i: (i, 0)),
        ],
        out_shape=[
            jax.ShapeDtypeStruct((2, N, 128), jnp.float32),
            jax.ShapeDtypeStruct((N, 128), jnp.float32),
        ],
    )(x, w1, degp)


def _tc_mid_body(dn, s_ref, hs_ref, dinv_ref, b_ref, w_ref, o_ref):
    d = dinv_ref[...]
    z0 = jax.nn.relu((s_ref[0] + hs_ref[0]) * d + b_ref[0, :128])
    z1 = jax.nn.relu((s_ref[1] + hs_ref[1]) * d + b_ref[0, 128:])
    z = jnp.concatenate([z0, z1], axis=1)
    h = jnp.dot(z, w_ref[...], preferred_element_type=jnp.float32)
    hs = h * d[:, 0:1]
    if dn == D:
        o_ref[0] = hs[:, :128]
        o_ref[1] = hs[:, 128:]
    else:
        o_ref[...] = hs


def _tc_mid(s_part, hs_prev, dinv_b, b_vec, w_next):
    dn = w_next.shape[1]
    if dn == D:
        out_spec = pl.BlockSpec((2, RB, 128), lambda i: (0, i, 0))
        out_shape = jax.ShapeDtypeStruct((2, N, 128), jnp.float32)
    else:
        out_spec = pl.BlockSpec((RB, dn), lambda i: (i, 0))
        out_shape = jax.ShapeDtypeStruct((N, dn), jnp.float32)
    return pl.pallas_call(
        functools.partial(_tc_mid_body, dn),
        grid=(GRID,),
        in_specs=[
            pl.BlockSpec((2, RB, 128), lambda i: (0, i, 0)),
            pl.BlockSpec((2, RB, 128), lambda i: (0, i, 0)),
            pl.BlockSpec((RB, 128), lambda i: (i, 0)),
            pl.BlockSpec((1, D), lambda i: (0, 0)),
            pl.BlockSpec((D, dn), lambda i: (0, 0)),
        ],
        out_specs=out_spec,
        out_shape=out_shape,
    )(s_part, hs_prev, dinv_b, b_vec, w_next)


def _tc_out_body(s_ref, hs_ref, dinv_ref, b_ref, o_ref):
    val = (s_ref[0] + s_ref[1] + hs_ref[...]) * dinv_ref[:, 0:1]
    o_ref[...] = val + b_ref[0, :]


def _tc_out(s3, hs3, dinv_b, b3):
    return pl.pallas_call(
        _tc_out_body,
        grid=(GRID,),
        in_specs=[
            pl.BlockSpec((2, RB, C), lambda i: (0, i, 0)),
            pl.BlockSpec((RB, C), lambda i: (i, 0)),
            pl.BlockSpec((RB, 128), lambda i: (i, 0)),
            pl.BlockSpec((1, C), lambda i: (0, 0)),
        ],
        out_specs=pl.BlockSpec((RB, C), lambda i: (i, 0)),
        out_shape=jax.ShapeDtypeStruct((N, C), jnp.float32),
    )(s3, hs3, dinv_b, b3)


def kernel(x, edge_index, W1, b1, W2, b2, W3, b3):
    src = edge_index[0].astype(jnp.int32)
    dst = edge_index[1].astype(jnp.int32)
    pad = EPAD - E
    src_p = jnp.concatenate([src, jnp.zeros((pad,), jnp.int32)])
    dst_p = jnp.concatenate([dst, jnp.full((pad,), N, jnp.int32)])
    src_fs = jnp.stack([src_p, src_p + N]).reshape(2, 16, EPAD // 16 // EB, EB)
    dst_fs = dst_p.reshape(16, EPAD // 16 // EB, EB)
    src_es = src_p.reshape(32, EPAD // 32 // EB, EB)
    dst_es = dst_p.reshape(32, EPAD // 32 // EB, EB)

    ones_hbm = jnp.ones((EB, 128), jnp.float32)
    zeros_deg = jnp.zeros((RPT_DEG, 128), jnp.float32)
    zeros64 = jnp.zeros((RPT_DEG, C), jnp.float32)

    degp = _make_deg()(dst_es, ones_hbm, zeros_deg)

    b1r = b1.reshape(1, D)
    b2r = b2.reshape(1, D)
    b3r = b3.reshape(1, C)


    spmm_fs = _make_spmm(128, False)
    spmm_es = _make_spmm(C, True)

    hs1, dinv_b = _tc1(x, W1, degp)
    s1 = spmm_fs(hs1.reshape(2 * N, 128), src_fs, dst_fs, zeros_deg)
    hs2 = _tc_mid(s1, hs1, dinv_b, b1r, W2)
    s2 = spmm_fs(hs2.reshape(2 * N, 128), src_fs, dst_fs, zeros_deg)
    hs3 = _tc_mid(s2, hs2, dinv_b, b2r, W3)
    s3 = spmm_es(hs3, src_es, dst_es, zeros64)
    return _tc_out(s3, hs3, dinv_b, b3r)


# speedup vs baseline: 1.0319x; 1.0032x over previous
import functools

import jax
import jax.numpy as jnp
from jax import lax
from jax.experimental import pallas as pl
from jax.experimental.pallas import tpu as pltpu
from jax.experimental.pallas import tpu_sc as plsc

N = 10000
E = 160000
D = 256
C = 64
EB = 128
EPAD = 163840
NPAD = 10240
RPT_DEG = NPAD // 16


@functools.cache
def _make_deg():
    mesh = plsc.VectorSubcoreMesh(core_axis_name="c", subcore_axis_name="s")
    nb = EPAD // 32 // EB

    @functools.partial(
        pl.kernel,
        out_type=jax.ShapeDtypeStruct((2, NPAD, 128), jnp.float32),
        mesh=mesh,
        scratch_types=[
            pltpu.VMEM((nb, EB), jnp.int32),
            pltpu.VMEM((EB, 128), jnp.float32),
            pltpu.VMEM_SHARED((NPAD, 128), jnp.float32),
        ],
    )
    def deg_kernel(dst_hbm, ones_hbm, zeros_hbm, out_hbm, dst_v, ones_v, acc):
        c = lax.axis_index("c")
        s = lax.axis_index("s")
        wid = s * 2 + c
        pltpu.sync_copy(dst_hbm.at[wid], dst_v)
        pltpu.sync_copy(ones_hbm, ones_v)
        pltpu.sync_copy(zeros_hbm, acc.at[pl.ds(s * RPT_DEG, RPT_DEG)])
        plsc.subcore_barrier()

        def body(b, carry):
            pltpu.sync_copy(ones_v, acc.at[dst_v.at[b]], add=True)
            return carry

        lax.fori_loop(0, nb, body, 0)
        plsc.subcore_barrier()
        sl = pl.ds(s * RPT_DEG, RPT_DEG)
        pltpu.sync_copy(acc.at[sl], out_hbm.at[c, sl])

    return deg_kernel


@functools.cache
def _make_spmm(W, edge_split):
    nb = EPAD // (32 if edge_split else 16) // EB
    mesh = plsc.VectorSubcoreMesh(core_axis_name="c", subcore_axis_name="s")

    @functools.partial(
        pl.kernel,
        out_type=jax.ShapeDtypeStruct((2, NPAD, W), jnp.float32),
        mesh=mesh,
        scratch_types=[
            pltpu.VMEM((nb, EB), jnp.int32),
            pltpu.VMEM((nb, EB), jnp.int32),
            pltpu.VMEM((EB, W), jnp.float32),
            pltpu.VMEM_SHARED((NPAD, W), jnp.float32),
        ],
        compiler_params=pltpu.CompilerParams(use_tc_tiling_on_sc=(W == 128)),
    )
    def spmm(hs_hbm, src_hbm, dst_hbm, zeros_hbm, out_hbm, src_v, dst_v, buf,
             acc):
        c = lax.axis_index("c")
        s = lax.axis_index("s")
        if edge_split:
            wid = s * 2 + c
            pltpu.sync_copy(src_hbm.at[wid], src_v)
            pltpu.sync_copy(dst_hbm.at[wid], dst_v)
        else:
            pltpu.sync_copy(src_hbm.at[c, s], src_v)
            pltpu.sync_copy(dst_hbm.at[s], dst_v)
        sl = pl.ds(s * RPT_DEG, RPT_DEG)
        pltpu.sync_copy(zeros_hbm, acc.at[sl])
        plsc.subcore_barrier()

        def body(b, carry):
            pltpu.sync_copy(hs_hbm.at[src_v.at[b]], buf)
            pltpu.sync_copy(buf, acc.at[dst_v.at[b]], add=True)
            return carry

        lax.fori_loop(0, nb, body, 0)
        plsc.subcore_barrier()
        pltpu.sync_copy(acc.at[sl], out_hbm.at[c, sl])

    return spmm


RB = 1000
GRID = N // RB


def _tc1_body(x_ref, w_ref, degp_ref, o_ref, dinv_ref):
    dinv = lax.rsqrt(degp_ref[0] + degp_ref[1] + 1.0)
    dinv_ref[...] = dinv
    h = jnp.dot(x_ref[...], w_ref[...], preferred_element_type=jnp.float32)
    hs = h * dinv[:, 0:1]
    o_ref[0] = hs[:, :128]
    o_ref[1] = hs[:, 128:]


def _tc1(x, w1, degp):
    return pl.pallas_call(
        _tc1_body,
        grid=(GRID,),
        in_specs=[
            pl.BlockSpec((RB, D), lambda i: (i, 0)),
            pl.BlockSpec((D, D), lambda i: (0, 0)),
            pl.BlockSpec((2, RB, 128), lambda i: (0, i, 0)),
        ],
        out_specs=[
            pl.BlockSpec((2, RB, 128), lambda i: (0, i, 0)),
            pl.BlockSpec((RB, 128), lambda i: (i, 0)),
        ],
        out_shape=[
            jax.ShapeDtypeStruct((2, N, 128), jnp.float32),
            jax.ShapeDtypeStruct((N, 128), jnp.float32),
        ],
    )(x, w1, degp)


def _tc_mid_body(dn, s_ref, hs_ref, dinv_ref, b_ref, w_ref, o_ref):
    d = dinv_ref[...]
    z0 = jax.nn.relu((s_ref[0] + hs_ref[0]) * d + b_ref[0, :128])
    z1 = jax.nn.relu((s_ref[1] + hs_ref[1]) * d + b_ref[0, 128:])
    z = jnp.concatenate([z0, z1], axis=1)
    h = jnp.dot(z, w_ref[...], preferred_element_type=jnp.float32)
    hs = h * d[:, 0:1]
    if dn == D:
        o_ref[0] = hs[:, :128]
        o_ref[1] = hs[:, 128:]
    else:
        o_ref[...] = hs


def _tc_mid(s_part, hs_prev, dinv_b, b_vec, w_next):
    dn = w_next.shape[1]
    if dn == D:
        out_spec = pl.BlockSpec((2, RB, 128), lambda i: (0, i, 0))
        out_shape = jax.ShapeDtypeStruct((2, N, 128), jnp.float32)
    else:
        out_spec = pl.BlockSpec((RB, dn), lambda i: (i, 0))
        out_shape = jax.ShapeDtypeStruct((N, dn), jnp.float32)
    return pl.pallas_call(
        functools.partial(_tc_mid_body, dn),
        grid=(GRID,),
        in_specs=[
            pl.BlockSpec((2, RB, 128), lambda i: (0, i, 0)),
            pl.BlockSpec((2, RB, 128), lambda i: (0, i, 0)),
            pl.BlockSpec((RB, 128), lambda i: (i, 0)),
            pl.BlockSpec((1, D), lambda i: (0, 0)),
            pl.BlockSpec((D, dn), lambda i: (0, 0)),
        ],
        out_specs=out_spec,
        out_shape=out_shape,
    )(s_part, hs_prev, dinv_b, b_vec, w_next)


def _tc_out_body(s_ref, hs_ref, dinv_ref, b_ref, o_ref):
    val = (s_ref[0] + s_ref[1] + hs_ref[...]) * dinv_ref[:, 0:1]
    o_ref[...] = val + b_ref[0, :]


def _tc_out(s3, hs3, dinv_b, b3):
    return pl.pallas_call(
        _tc_out_body,
        grid=(GRID,),
        in_specs=[
            pl.BlockSpec((2, RB, C), lambda i: (0, i, 0)),
            pl.BlockSpec((RB, C), lambda i: (i, 0)),
            pl.BlockSpec((RB, 128), lambda i: (i, 0)),
            pl.BlockSpec((1, C), lambda i: (0, 0)),
        ],
        out_specs=pl.BlockSpec((RB, C), lambda i: (i, 0)),
        out_shape=jax.ShapeDtypeStruct((N, C), jnp.float32),
    )(s3, hs3, dinv_b, b3)


def kernel(x, edge_index, W1, b1, W2, b2, W3, b3):
    src = edge_index[0].astype(jnp.int32)
    dst = edge_index[1].astype(jnp.int32)
    pad = EPAD - E
    src_p = jnp.concatenate([src, jnp.zeros((pad,), jnp.int32)])
    dst_p = jnp.concatenate([dst, jnp.full((pad,), N, jnp.int32)])
    src_fs = jnp.stack([src_p, src_p + N]).reshape(2, 16, EPAD // 16 // EB, EB)
    dst_fs = dst_p.reshape(16, EPAD // 16 // EB, EB)
    src_es = src_p.reshape(32, EPAD // 32 // EB, EB)
    dst_es = dst_p.reshape(32, EPAD // 32 // EB, EB)

    ones_hbm = jnp.ones((EB, 128), jnp.float32)
    zeros_deg = jnp.zeros((RPT_DEG, 128), jnp.float32)
    zeros64 = jnp.zeros((RPT_DEG, C), jnp.float32)

    degp = _make_deg()(dst_es, ones_hbm, zeros_deg)

    b1r = b1.reshape(1, D)
    b2r = b2.reshape(1, D)
    b3r = b3.reshape(1, C)


    spmm_fs = _make_spmm(128, False)
    spmm_es = _make_spmm(C, True)

    hs1, dinv_b = _tc1(x, W1, degp)
    s1 = spmm_fs(hs1.reshape(2 * N, 128), src_fs, dst_fs, zeros_deg)
    hs2 = _tc_mid(s1, hs1, dinv_b, b1r, W2)
    s2 = spmm_fs(hs2.reshape(2 * N, 128), src_fs, dst_fs, zeros_deg)
    hs3 = _tc_mid(s2, hs2, dinv_b, b2r, W3)
    s3 = spmm_es(hs3, src_es, dst_es, zeros64)
    return _tc_out(s3, hs3, dinv_b, b3r)


# speedup vs baseline: 1.0528x; 1.0202x over previous
import functools

import jax
import jax.numpy as jnp
from jax import lax
from jax.experimental import pallas as pl
from jax.experimental.pallas import tpu as pltpu
from jax.experimental.pallas import tpu_sc as plsc

N = 10000
E = 160000
D = 256
C = 64
EB = 128
EPAD = 163840
NPAD = 10240
RPT_DEG = NPAD // 16


@functools.cache
def _make_deg():
    mesh = plsc.VectorSubcoreMesh(core_axis_name="c", subcore_axis_name="s")
    epw = EPAD // 32

    @functools.partial(
        pl.kernel,
        out_type=jax.ShapeDtypeStruct((2, NPAD, 16), jnp.float32),
        mesh=mesh,
        scratch_types=[
            pltpu.VMEM((epw,), jnp.int32),
            pltpu.VMEM((NPAD,), jnp.float32),
            pltpu.VMEM((16, RPT_DEG), jnp.float32),
            pltpu.VMEM((RPT_DEG, 16), jnp.float32),
            pltpu.VMEM_SHARED((16, 16, RPT_DEG), jnp.float32),
        ],
        compiler_params=pltpu.CompilerParams(use_tc_tiling_on_sc=False,
                                             needs_layout_passes=False),
    )
    def deg_kernel(dst_hbm, zeros_hbm, out_hbm, dst_v, hist, merge_v, bcast_v,
                   stage):
        c = lax.axis_index("c")
        s = lax.axis_index("s")
        wid = s * 2 + c
        pltpu.sync_copy(dst_hbm.at[wid], dst_v)
        pltpu.sync_copy(zeros_hbm, hist)
        ones16 = jnp.full((16,), 1.0, jnp.float32)

        def body(i, carry):
            idx = dst_v[pl.ds(i * 16, 16)]
            plsc.addupdate_scatter(hist, [idx], ones16)
            return carry

        lax.fori_loop(0, epw // 16, body, 0)
        for t in range(16):
            pltpu.sync_copy(hist.at[pl.ds(t * RPT_DEG, RPT_DEG)],
                            stage.at[t, s])
        plsc.subcore_barrier()
        pltpu.sync_copy(stage.at[s], merge_v)

        def merge(j, carry):
            tot = jnp.zeros((16,), jnp.float32)
            for t in range(16):
                tot = tot + merge_v[t, pl.ds(j * 16, 16)]
            r0 = j * 16
            for u in range(16):
                bcast_v[r0 + u, :] = jnp.broadcast_to(tot[u], (16,))
            return carry

        lax.fori_loop(0, RPT_DEG // 16, merge, 0)
        sl = pl.ds(s * RPT_DEG, RPT_DEG)
        pltpu.sync_copy(bcast_v, out_hbm.at[c, sl])

    return deg_kernel


@functools.cache
def _make_spmm(W, edge_split):
    nb = EPAD // (32 if edge_split else 16) // EB
    mesh = plsc.VectorSubcoreMesh(core_axis_name="c", subcore_axis_name="s")

    @functools.partial(
        pl.kernel,
        out_type=jax.ShapeDtypeStruct((2, NPAD, W), jnp.float32),
        mesh=mesh,
        scratch_types=[
            pltpu.VMEM((nb, EB), jnp.int32),
            pltpu.VMEM((nb, EB), jnp.int32),
            pltpu.VMEM((EB, W), jnp.float32),
            pltpu.VMEM_SHARED((NPAD, W), jnp.float32),
        ],
        compiler_params=pltpu.CompilerParams(use_tc_tiling_on_sc=(W == 128)),
    )
    def spmm(hs_hbm, src_hbm, dst_hbm, zeros_hbm, out_hbm, src_v, dst_v, buf,
             acc):
        c = lax.axis_index("c")
        s = lax.axis_index("s")
        if edge_split:
            wid = s * 2 + c
            pltpu.sync_copy(src_hbm.at[wid], src_v)
            pltpu.sync_copy(dst_hbm.at[wid], dst_v)
        else:
            pltpu.sync_copy(src_hbm.at[c, s], src_v)
            pltpu.sync_copy(dst_hbm.at[s], dst_v)
        sl = pl.ds(s * RPT_DEG, RPT_DEG)
        pltpu.sync_copy(zeros_hbm, acc.at[sl])
        plsc.subcore_barrier()

        def body(b, carry):
            pltpu.sync_copy(hs_hbm.at[src_v.at[b]], buf)
            pltpu.sync_copy(buf, acc.at[dst_v.at[b]], add=True)
            return carry

        lax.fori_loop(0, nb, body, 0)
        plsc.subcore_barrier()
        pltpu.sync_copy(acc.at[sl], out_hbm.at[c, sl])

    return spmm


RB = 1000
GRID = N // RB


def _tc1_body(x_ref, w_ref, degp_ref, o_ref, dinv_ref):
    dcol = lax.rsqrt(degp_ref[0, :, 0:1] + degp_ref[1, :, 0:1] + 1.0)
    dinv_ref[...] = jnp.broadcast_to(dcol, dinv_ref.shape)
    h = jnp.dot(x_ref[...], w_ref[...], preferred_element_type=jnp.float32)
    hs = h * dcol
    o_ref[0] = hs[:, :128]
    o_ref[1] = hs[:, 128:]


def _tc1(x, w1, degp):
    return pl.pallas_call(
        _tc1_body,
        grid=(GRID,),
        in_specs=[
            pl.BlockSpec((RB, D), lambda i: (i, 0)),
            pl.BlockSpec((D, D), lambda i: (0, 0)),
            pl.BlockSpec((2, RB, 16), lambda i: (0, i, 0)),
        ],
        out_specs=[
            pl.BlockSpec((2, RB, 128), lambda i: (0, i, 0)),
            pl.BlockSpec((RB, 128), lambda i: (i, 0)),
        ],
        out_shape=[
            jax.ShapeDtypeStruct((2, N, 128), jnp.float32),
            jax.ShapeDtypeStruct((N, 128), jnp.float32),
        ],
    )(x, w1, degp)


def _tc_mid_body(dn, s_ref, hs_ref, dinv_ref, b_ref, w_ref, o_ref):
    d = dinv_ref[...]
    z0 = jax.nn.relu((s_ref[0] + hs_ref[0]) * d + b_ref[0, :128])
    z1 = jax.nn.relu((s_ref[1] + hs_ref[1]) * d + b_ref[0, 128:])
    z = jnp.concatenate([z0, z1], axis=1)
    h = jnp.dot(z, w_ref[...], preferred_element_type=jnp.float32)
    hs = h * d[:, 0:1]
    if dn == D:
        o_ref[0] = hs[:, :128]
        o_ref[1] = hs[:, 128:]
    else:
        o_ref[...] = hs


def _tc_mid(s_part, hs_prev, dinv_b, b_vec, w_next):
    dn = w_next.shape[1]
    if dn == D:
        out_spec = pl.BlockSpec((2, RB, 128), lambda i: (0, i, 0))
        out_shape = jax.ShapeDtypeStruct((2, N, 128), jnp.float32)
    else:
        out_spec = pl.BlockSpec((RB, dn), lambda i: (i, 0))
        out_shape = jax.ShapeDtypeStruct((N, dn), jnp.float32)
    return pl.pallas_call(
        functools.partial(_tc_mid_body, dn),
        grid=(GRID,),
        in_specs=[
            pl.BlockSpec((2, RB, 128), lambda i: (0, i, 0)),
            pl.BlockSpec((2, RB, 128), lambda i: (0, i, 0)),
            pl.BlockSpec((RB, 128), lambda i: (i, 0)),
            pl.BlockSpec((1, D), lambda i: (0, 0)),
            pl.BlockSpec((D, dn), lambda i: (0, 0)),
        ],
        out_specs=out_spec,
        out_shape=out_shape,
    )(s_part, hs_prev, dinv_b, b_vec, w_next)


def _tc_out_body(s_ref, hs_ref, dinv_ref, b_ref, o_ref):
    val = (s_ref[0] + s_ref[1] + hs_ref[...]) * dinv_ref[:, 0:1]
    o_ref[...] = val + b_ref[0, :]


def _tc_out(s3, hs3, dinv_b, b3):
    return pl.pallas_call(
        _tc_out_body,
        grid=(GRID,),
        in_specs=[
            pl.BlockSpec((2, RB, C), lambda i: (0, i, 0)),
            pl.BlockSpec((RB, C), lambda i: (i, 0)),
            pl.BlockSpec((RB, 128), lambda i: (i, 0)),
            pl.BlockSpec((1, C), lambda i: (0, 0)),
        ],
        out_specs=pl.BlockSpec((RB, C), lambda i: (i, 0)),
        out_shape=jax.ShapeDtypeStruct((N, C), jnp.float32),
    )(s3, hs3, dinv_b, b3)


def kernel(x, edge_index, W1, b1, W2, b2, W3, b3):
    src = edge_index[0].astype(jnp.int32)
    dst = edge_index[1].astype(jnp.int32)
    pad = EPAD - E
    src_p = jnp.concatenate([src, jnp.zeros((pad,), jnp.int32)])
    dst_p = jnp.concatenate([dst, jnp.full((pad,), N, jnp.int32)])
    src_fs = jnp.stack([src_p, src_p + N]).reshape(2, 16, EPAD // 16 // EB, EB)
    dst_fs = dst_p.reshape(16, EPAD // 16 // EB, EB)
    src_es = src_p.reshape(32, EPAD // 32 // EB, EB)
    dst_es = dst_p.reshape(32, EPAD // 32 // EB, EB)

    zeros_hist = jnp.zeros((NPAD,), jnp.float32)
    zeros_deg = jnp.zeros((RPT_DEG, 128), jnp.float32)
    dst_deg = dst_p.reshape(32, EPAD // 32)
    zeros64 = jnp.zeros((RPT_DEG, C), jnp.float32)

    degp = _make_deg()(dst_deg, zeros_hist)

    b1r = b1.reshape(1, D)
    b2r = b2.reshape(1, D)
    b3r = b3.reshape(1, C)


    spmm_fs = _make_spmm(128, False)
    spmm_es = _make_spmm(C, True)

    hs1, dinv_b = _tc1(x, W1, degp)
    s1 = spmm_fs(hs1.reshape(2 * N, 128), src_fs, dst_fs, zeros_deg)
    hs2 = _tc_mid(s1, hs1, dinv_b, b1r, W2)
    s2 = spmm_fs(hs2.reshape(2 * N, 128), src_fs, dst_fs, zeros_deg)
    hs3 = _tc_mid(s2, hs2, dinv_b, b2r, W3)
    s3 = spmm_es(hs3, src_es, dst_es, zeros64)
    return _tc_out(s3, hs3, dinv_b, b3r)
